# Initial kernel scaffold; baseline (speedup 1.0000x reference)
import jax
import jax.numpy as jnp
from jax.experimental import pallas as pl


def kernel(features, edge_index, precomp, connection, W11, W12, S1, W21, W22, S2, W31, W32, S3, lin_w, lin_b):
    raise NotImplementedError("write your pallas kernel here")


# speedup vs baseline: 2.7265x; 2.7265x over previous
import functools

import jax
import jax.numpy as jnp
from jax import lax
from jax.experimental import pallas as pl
from jax.experimental.pallas import tpu as pltpu
from jax.experimental.pallas import tpu_sc as plsc

N_NODES = 50000
N_EDGES = 800000
NLAYERS = 5
KB = 6

NC, NS, CH = 2, 16, 128
NW = NC * NS
CPW = 196
EW = CPW * CH
EP = NW * EW
NPAD = 51200
ROWS_PT = NPAD // NS
GK = 7
GE = GK * CH
NGRP = CPW // GK

_mesh = plsc.VectorSubcoreMesh(core_axis_name="c", subcore_axis_name="s")
_sc_params = pltpu.CompilerParams(use_tc_tiling_on_sc=False)


def _make_gather(D):
    @functools.partial(
        pl.kernel,
        mesh=_mesh,
        compiler_params=_sc_params,
        out_type=jax.ShapeDtypeStruct((EP, D), jnp.float32),
        scratch_types=[
            pltpu.VMEM((2, GK, CH), jnp.int32),
            pltpu.VMEM((2, GE, D), jnp.float32),
            pltpu.SemaphoreType.DMA,
            pltpu.SemaphoreType.DMA,
        ],
    )
    def gk(table, src2d, out, idx2, buf2, sem_g, sem_o):
        cid = lax.axis_index("c")
        sid = lax.axis_index("s")
        wid = sid * NC + cid
        rows = src2d.at[wid]

        def load_idx(g, cur):
            pltpu.sync_copy(rows.at[pl.ds(g * GK, GK)], idx2.at[cur])

        def fire_group(cur):
            for j in range(GK):
                pltpu.async_copy(
                    table.at[idx2.at[cur].at[j]],
                    buf2.at[cur].at[pl.ds(j * CH, CH)], sem_g)

        load_idx(0, 0)
        fire_group(0)

        def body(t, carry):
            for cur in range(2):
                g = 2 * t + cur
                for j in range(GK):
                    pltpu.make_async_copy(
                        table.at[idx2.at[cur].at[j]],
                        buf2.at[cur].at[pl.ds(j * CH, CH)], sem_g).wait()

                @pl.when(g >= 1)
                def _():
                    pltpu.make_async_copy(
                        buf2.at[1 - cur], out.at[pl.ds(0, GE)], sem_o).wait()

                woff = pl.multiple_of(wid * EW + g * GE, GE)
                pltpu.async_copy(buf2.at[cur], out.at[pl.ds(woff, GE)], sem_o)

                @pl.when(g + 1 < NGRP)
                def _():
                    load_idx(g + 1, 1 - cur)
                    fire_group(1 - cur)
            return carry

        lax.fori_loop(0, NGRP // 2, body, 0)
        pltpu.make_async_copy(
            buf2.at[1], out.at[pl.ds(0, GE)], sem_o).wait()

    return gk


def _make_scatter(O):
    @functools.partial(
        pl.kernel,
        mesh=_mesh,
        compiler_params=_sc_params,
        out_type=jax.ShapeDtypeStruct((NC * NPAD, O), jnp.float32),
        scratch_types=[
            pltpu.VMEM_SHARED((NPAD, O), jnp.float32),
            pltpu.VMEM((2, GK, CH), jnp.int32),
            pltpu.VMEM((2, GE, O), jnp.float32),
            pltpu.SemaphoreType.DMA,
        ],
    )
    def sk(msg, dst2d, zer, out, accum, idx2, mb2, sem_m):
        cid = lax.axis_index("c")
        sid = lax.axis_index("s")
        wid = sid * NC + cid
        r0 = pl.multiple_of(sid * ROWS_PT, ROWS_PT)
        pltpu.sync_copy(zer.at[pl.ds(r0, ROWS_PT)], accum.at[pl.ds(r0, ROWS_PT)])
        irows = dst2d.at[wid]
        pltpu.sync_copy(irows.at[pl.ds(0, GK)], idx2.at[0])
        plsc.subcore_barrier()

        def fire_load(g, cur):
            loff = pl.multiple_of(wid * EW + g * GE, GE)
            pltpu.async_copy(msg.at[pl.ds(loff, GE)], mb2.at[cur], sem_m)

        fire_load(0, 0)

        def body(t, carry):
            for cur in range(2):
                g = 2 * t + cur
                pltpu.make_async_copy(
                    msg.at[pl.ds(0, GE)], mb2.at[cur], sem_m).wait()

                @pl.when(g + 1 < NGRP)
                def _():
                    fire_load(g + 1, 1 - cur)
                    pltpu.sync_copy(
                        irows.at[pl.ds((g + 1) * GK, GK)], idx2.at[1 - cur])

                for j in range(GK):
                    pltpu.sync_copy(
                        mb2.at[cur].at[pl.ds(j * CH, CH)],
                        accum.at[idx2.at[cur].at[j]], add=True)
            return carry

        lax.fori_loop(0, NGRP // 2, body, 0)
        plsc.subcore_barrier()
        off = pl.multiple_of(cid * NPAD + r0, ROWS_PT)
        pltpu.sync_copy(accum.at[pl.ds(r0, ROWS_PT)], out.at[pl.ds(off, ROWS_PT)])

    return sk


def _edge_mm(Cin, din, O, BE=2048):
    I = Cin * din
    KO = KB * O

    def body(xs_ref, cf_ref, w_ref, out_ref):
        x = xs_ref[...]
        cf = cf_ref[...]
        if din == 3:
            craw = cf[:, 6:7]
            sraw = cf[:, 7:8]
            nrm = jnp.sqrt(craw * craw + sraw * sraw) + 1e-8
            c = craw / nrm
            s = sraw / nrm
            x0 = x[:, :Cin]
            x1 = x[:, Cin:2 * Cin]
            x2 = x[:, 2 * Cin:]
            x = jnp.concatenate([x0, x1 * c - x2 * s, x1 * s + x2 * c], axis=1)
        t = jnp.dot(x, w_ref[...], preferred_element_type=jnp.float32)
        msg = cf[:, 0:1] * t[:, :O]
        for k in range(1, KB):
            msg = msg + cf[:, k:k + 1] * t[:, k * O:(k + 1) * O]
        out_ref[...] = msg

    return pl.pallas_call(
        body,
        grid=(EP // BE,),
        in_specs=[
            pl.BlockSpec((BE, I), lambda i: (i, 0)),
            pl.BlockSpec((BE, 8), lambda i: (i, 0)),
            pl.BlockSpec((I, KO), lambda i: (0, 0)),
        ],
        out_specs=pl.BlockSpec((BE, O), lambda i: (i, 0)),
        out_shape=jax.ShapeDtypeStruct((EP, O), jnp.float32),
    )


def _node_relu_sum(O, BN=400):
    def body(p0_ref, p1_ref, out_ref):
        out_ref[...] = jnp.maximum(p0_ref[...] + p1_ref[...], 0.0)

    return pl.pallas_call(
        body,
        grid=(N_NODES // BN,),
        in_specs=[
            pl.BlockSpec((BN, O), lambda i: (i, 0)),
            pl.BlockSpec((BN, O), lambda i: (i + NPAD // BN, 0)),
        ],
        out_specs=pl.BlockSpec((BN, O), lambda i: (i, 0)),
        out_shape=jax.ShapeDtypeStruct((N_NODES, O), jnp.float32),
    )


def _node_shortcut(O, Cp, BN=400):
    def body(p0_ref, p1_ref, xp_ref, s_ref, out_ref):
        sc = jnp.dot(xp_ref[...], s_ref[...], preferred_element_type=jnp.float32)
        out_ref[...] = jnp.maximum(p0_ref[...] + p1_ref[...] + sc, 0.0)

    return pl.pallas_call(
        body,
        grid=(N_NODES // BN,),
        in_specs=[
            pl.BlockSpec((BN, O), lambda i: (i, 0)),
            pl.BlockSpec((BN, O), lambda i: (i + NPAD // BN, 0)),
            pl.BlockSpec((BN, Cp), lambda i: (i, 0)),
            pl.BlockSpec((Cp, O), lambda i: (0, 0)),
        ],
        out_specs=pl.BlockSpec((BN, O), lambda i: (i, 0)),
        out_shape=jax.ShapeDtypeStruct((N_NODES, O), jnp.float32),
    )


def _node_final(BN=400):
    def body(p0_ref, p1_ref, xp_ref, s_ref, lwb_ref, out_ref):
        sc = jnp.dot(xp_ref[...], s_ref[...], preferred_element_type=jnp.float32)
        x3 = jnp.maximum(p0_ref[...] + p1_ref[...] + sc, 0.0)
        y = jnp.sum(x3 * lwb_ref[0:1, 0:8], axis=1, keepdims=True) + lwb_ref[0, 8]
        out_ref[...] = jnp.maximum(y, 0.0)

    return pl.pallas_call(
        body,
        grid=(N_NODES // BN,),
        in_specs=[
            pl.BlockSpec((BN, 8), lambda i: (i, 0)),
            pl.BlockSpec((BN, 8), lambda i: (i + NPAD // BN, 0)),
            pl.BlockSpec((BN, 24), lambda i: (i, 0)),
            pl.BlockSpec((24, 8), lambda i: (0, 0)),
            pl.BlockSpec((1, 9), lambda i: (0, 0)),
        ],
        out_specs=pl.BlockSpec((BN, 1), lambda i: (i, 0)),
        out_shape=jax.ShapeDtypeStruct((N_NODES, 1), jnp.float32),
    )


def _wprep(W, Cout, dout, Cin, din):
    O = dout * Cout
    Wd = W.reshape(KB, Cout, dout, Cin, din).transpose(0, 2, 1, 4, 3)
    Wd = Wd.reshape(KB, O, din * Cin)
    return Wd.transpose(2, 0, 1).reshape(din * Cin, KB * O)


def kernel(features, edge_index, precomp, connection,
           W11, W12, S1, W21, W22, S2, W31, W32, S3, lin_w, lin_b):
    f32 = jnp.float32
    src = edge_index[0]
    dst = edge_index[1]
    pe = EP - N_EDGES
    src2d = jnp.concatenate([src, jnp.zeros((pe,), jnp.int32)]).reshape(NW, CPW, CH)
    dst2d = jnp.concatenate(
        [dst, jnp.full((pe,), N_NODES, jnp.int32)]).reshape(NW, CPW, CH)
    coef = jnp.concatenate([precomp.reshape(N_EDGES, KB), connection], axis=1)
    coef = jnp.concatenate([coef, jnp.zeros((pe, 8), f32)])
    zer24 = jnp.zeros((NPAD, 24), f32)
    zer8 = jnp.zeros((NPAD, 8), f32)

    Wm11 = _wprep(W11, 8, 3, 32, 1)
    Wm12 = _wprep(W12, 8, 3, 8, 3)
    Wm21 = _wprep(W21, 8, 3, 8, 3)
    Wm22 = _wprep(W22, 8, 3, 8, 3)
    Wm31 = _wprep(W31, 8, 1, 8, 3)
    Wm32 = _wprep(W32, 8, 1, 8, 1)
    S1pad = jnp.pad(S1.T, ((0, 0), (0, 16)))
    S2pad = jnp.zeros((24, 24), f32)
    for dd in range(3):
        S2pad = S2pad.at[dd * 8:(dd + 1) * 8, dd * 8:(dd + 1) * 8].set(S2.T)
    S3pad = jnp.concatenate([S3.T, jnp.zeros((16, 8), f32)], axis=0)
    lwb = jnp.concatenate([lin_w, lin_b[:, None]], axis=1)

    g32 = _make_gather(32)
    g24 = _make_gather(24)
    g8 = _make_gather(8)
    sc24 = _make_scatter(24)
    sc8 = _make_scatter(8)

    xs = g32(features, src2d)
    parts = sc24(_edge_mm(32, 1, 24)(xs, coef, Wm11), dst2d, zer24)
    h = _node_relu_sum(24)(parts, parts)
    xs = g24(h, src2d)
    parts = sc24(_edge_mm(8, 3, 24)(xs, coef, Wm12), dst2d, zer24)
    x1 = _node_shortcut(24, 32)(parts, parts, features, S1pad)
    xs = g24(x1, src2d)
    parts = sc24(_edge_mm(8, 3, 24)(xs, coef, Wm21), dst2d, zer24)
    h = _node_relu_sum(24)(parts, parts)
    xs = g24(h, src2d)
    parts = sc24(_edge_mm(8, 3, 24)(xs, coef, Wm22), dst2d, zer24)
    x2 = _node_shortcut(24, 24)(parts, parts, x1, S2pad)
    xs = g24(x2, src2d)
    parts = sc8(_edge_mm(8, 3, 8)(xs, coef, Wm31), dst2d, zer8)
    h = _node_relu_sum(8)(parts, parts)
    xs = g8(h, src2d)
    parts = sc8(_edge_mm(8, 1, 8)(xs, coef, Wm32), dst2d, zer8)
    y = _node_final()(parts, parts, x2, S3pad, lwb)
    return y[:, 0].reshape(NLAYERS, N_NODES // NLAYERS).T


# speedup vs baseline: 5.3482x; 1.9616x over previous
import functools

import jax
import jax.numpy as jnp
from jax import lax
from jax.experimental import pallas as pl
from jax.experimental.pallas import tpu as pltpu
from jax.experimental.pallas import tpu_sc as plsc

N_NODES = 50000
N_EDGES = 800000
NLAYERS = 5
KB = 6

NC, NS, CH = 2, 16, 128
NW = NC * NS
CPW = 196
EW = CPW * CH
EP = NW * EW
NPAD = 51200
ROWS_PT = NPAD // NS
GK = 7
GE = GK * CH
NGRP = CPW // GK

_mesh = plsc.VectorSubcoreMesh(core_axis_name="c", subcore_axis_name="s")
_sc_params = pltpu.CompilerParams(use_tc_tiling_on_sc=False)


def _make_gather(D):
    @functools.partial(
        pl.kernel,
        mesh=_mesh,
        compiler_params=_sc_params,
        out_type=jax.ShapeDtypeStruct((EP, D), jnp.float32),
        scratch_types=[
            pltpu.VMEM((2, GK, CH), jnp.int32),
            pltpu.VMEM((2, GE, D), jnp.float32),
            pltpu.SemaphoreType.DMA,
            pltpu.SemaphoreType.DMA,
        ],
    )
    def gk(table, src2d, out, idx2, buf2, sem_g, sem_o):
        cid = lax.axis_index("c")
        sid = lax.axis_index("s")
        wid = sid * NC + cid
        rows = src2d.at[wid]

        def load_idx(g, cur):
            pltpu.sync_copy(rows.at[pl.ds(g * GK, GK)], idx2.at[cur])

        def fire_group(cur):
            for j in range(GK):
                pltpu.async_copy(
                    table.at[idx2.at[cur].at[j]],
                    buf2.at[cur].at[pl.ds(j * CH, CH)], sem_g)

        load_idx(0, 0)
        fire_group(0)

        def body(t, carry):
            for cur in range(2):
                g = 2 * t + cur
                for j in range(GK):
                    pltpu.make_async_copy(
                        table.at[idx2.at[cur].at[j]],
                        buf2.at[cur].at[pl.ds(j * CH, CH)], sem_g).wait()

                @pl.when(g >= 1)
                def _():
                    pltpu.make_async_copy(
                        buf2.at[1 - cur], out.at[pl.ds(0, GE)], sem_o).wait()

                woff = pl.multiple_of(wid * EW + g * GE, GE)
                pltpu.async_copy(buf2.at[cur], out.at[pl.ds(woff, GE)], sem_o)

                @pl.when(g + 1 < NGRP)
                def _():
                    load_idx(g + 1, 1 - cur)
                    fire_group(1 - cur)
            return carry

        lax.fori_loop(0, NGRP // 2, body, 0)
        pltpu.make_async_copy(
            buf2.at[1], out.at[pl.ds(0, GE)], sem_o).wait()

    return gk


def _make_scatter(O):
    @functools.partial(
        pl.kernel,
        mesh=_mesh,
        compiler_params=_sc_params,
        out_type=jax.ShapeDtypeStruct((NC * NPAD, O), jnp.float32),
        scratch_types=[
            pltpu.VMEM_SHARED((NPAD, O), jnp.float32),
            pltpu.VMEM((2, GK, CH), jnp.int32),
            pltpu.VMEM((2, GE, O), jnp.float32),
            pltpu.SemaphoreType.DMA,
        ],
    )
    def sk(msg, dst2d, zer, out, accum, idx2, mb2, sem_m):
        cid = lax.axis_index("c")
        sid = lax.axis_index("s")
        wid = sid * NC + cid
        r0 = pl.multiple_of(sid * ROWS_PT, ROWS_PT)
        pltpu.sync_copy(zer.at[pl.ds(r0, ROWS_PT)], accum.at[pl.ds(r0, ROWS_PT)])
        irows = dst2d.at[wid]
        pltpu.sync_copy(irows.at[pl.ds(0, GK)], idx2.at[0])
        plsc.subcore_barrier()

        def fire_load(g, cur):
            loff = pl.multiple_of(wid * EW + g * GE, GE)
            pltpu.async_copy(msg.at[pl.ds(loff, GE)], mb2.at[cur], sem_m)

        fire_load(0, 0)

        def body(t, carry):
            for cur in range(2):
                g = 2 * t + cur
                pltpu.make_async_copy(
                    msg.at[pl.ds(0, GE)], mb2.at[cur], sem_m).wait()

                @pl.when(g + 1 < NGRP)
                def _():
                    fire_load(g + 1, 1 - cur)
                    pltpu.sync_copy(
                        irows.at[pl.ds((g + 1) * GK, GK)], idx2.at[1 - cur])

                for j in range(GK):
                    pltpu.sync_copy(
                        mb2.at[cur].at[pl.ds(j * CH, CH)],
                        accum.at[idx2.at[cur].at[j]], add=True)
            return carry

        lax.fori_loop(0, NGRP // 2, body, 0)
        plsc.subcore_barrier()
        off = pl.multiple_of(cid * NPAD + r0, ROWS_PT)
        pltpu.sync_copy(accum.at[pl.ds(r0, ROWS_PT)], out.at[pl.ds(off, ROWS_PT)])

    return sk


def _normalize_cs(BE=4096):
    def body(cf_ref, out_ref):
        cf = cf_ref[...]
        c = cf[6:7, :]
        s = cf[7:8, :]
        nrm = jnp.sqrt(c * c + s * s) + 1e-8
        out_ref[...] = jnp.concatenate([cf[0:6, :], c / nrm, s / nrm], axis=0)

    return pl.pallas_call(
        body,
        grid=(EP // BE,),
        in_specs=[pl.BlockSpec((8, BE), lambda i: (0, i))],
        out_specs=pl.BlockSpec((8, BE), lambda i: (0, i)),
        out_shape=jax.ShapeDtypeStruct((8, EP), jnp.float32),
    )


def _edge_mm(Cin, din, O, BE=2048):
    I = Cin * din
    KO = KB * O

    def body(xs_ref, cf_ref, w_ref, out_ref):
        cf = cf_ref[...]
        xt = xs_ref[...].T
        if din == 3:
            c = cf[6:7, :]
            s = cf[7:8, :]
            x0 = xt[:Cin, :]
            x1 = xt[Cin:2 * Cin, :]
            x2 = xt[2 * Cin:, :]
            xt = jnp.concatenate([x0, x1 * c - x2 * s, x1 * s + x2 * c],
                                 axis=0)
        tt = jnp.dot(w_ref[...], xt, preferred_element_type=jnp.float32)
        msg = cf[0:1, :] * tt[:O, :]
        for k in range(1, KB):
            msg = msg + cf[k:k + 1, :] * tt[k * O:(k + 1) * O, :]
        out_ref[...] = msg.T

    return pl.pallas_call(
        body,
        grid=(EP // BE,),
        in_specs=[
            pl.BlockSpec((BE, I), lambda i: (i, 0)),
            pl.BlockSpec((8, BE), lambda i: (0, i)),
            pl.BlockSpec((KO, I), lambda i: (0, 0)),
        ],
        out_specs=pl.BlockSpec((BE, O), lambda i: (i, 0)),
        out_shape=jax.ShapeDtypeStruct((EP, O), jnp.float32),
    )


def _node_relu_sum(O, BN=400):
    def body(p0_ref, p1_ref, out_ref):
        out_ref[...] = jnp.maximum(p0_ref[...] + p1_ref[...], 0.0)

    return pl.pallas_call(
        body,
        grid=(N_NODES // BN,),
        in_specs=[
            pl.BlockSpec((BN, O), lambda i: (i, 0)),
            pl.BlockSpec((BN, O), lambda i: (i + NPAD // BN, 0)),
        ],
        out_specs=pl.BlockSpec((BN, O), lambda i: (i, 0)),
        out_shape=jax.ShapeDtypeStruct((N_NODES, O), jnp.float32),
    )


def _node_shortcut(O, Cp, BN=400):
    def body(p0_ref, p1_ref, xp_ref, s_ref, out_ref):
        sc = jnp.dot(xp_ref[...], s_ref[...], preferred_element_type=jnp.float32)
        out_ref[...] = jnp.maximum(p0_ref[...] + p1_ref[...] + sc, 0.0)

    return pl.pallas_call(
        body,
        grid=(N_NODES // BN,),
        in_specs=[
            pl.BlockSpec((BN, O), lambda i: (i, 0)),
            pl.BlockSpec((BN, O), lambda i: (i + NPAD // BN, 0)),
            pl.BlockSpec((BN, Cp), lambda i: (i, 0)),
            pl.BlockSpec((Cp, O), lambda i: (0, 0)),
        ],
        out_specs=pl.BlockSpec((BN, O), lambda i: (i, 0)),
        out_shape=jax.ShapeDtypeStruct((N_NODES, O), jnp.float32),
    )


def _node_final(BN=400):
    def body(p0_ref, p1_ref, xp_ref, s_ref, lwb_ref, out_ref):
        sc = jnp.dot(xp_ref[...], s_ref[...], preferred_element_type=jnp.float32)
        x3 = jnp.maximum(p0_ref[...] + p1_ref[...] + sc, 0.0)
        y = jnp.sum(x3 * lwb_ref[0:1, 0:8], axis=1, keepdims=True) + lwb_ref[0, 8]
        out_ref[...] = jnp.maximum(y, 0.0)

    return pl.pallas_call(
        body,
        grid=(N_NODES // BN,),
        in_specs=[
            pl.BlockSpec((BN, 8), lambda i: (i, 0)),
            pl.BlockSpec((BN, 8), lambda i: (i + NPAD // BN, 0)),
            pl.BlockSpec((BN, 24), lambda i: (i, 0)),
            pl.BlockSpec((24, 8), lambda i: (0, 0)),
            pl.BlockSpec((1, 9), lambda i: (0, 0)),
        ],
        out_specs=pl.BlockSpec((BN, 1), lambda i: (i, 0)),
        out_shape=jax.ShapeDtypeStruct((N_NODES, 1), jnp.float32),
    )


def _wprep(W, Cout, dout, Cin, din):
    O = dout * Cout
    Wd = W.reshape(KB, Cout, dout, Cin, din).transpose(0, 2, 1, 4, 3)
    return Wd.reshape(KB * O, din * Cin)


def kernel(features, edge_index, precomp, connection,
           W11, W12, S1, W21, W22, S2, W31, W32, S3, lin_w, lin_b):
    f32 = jnp.float32
    src = edge_index[0]
    dst = edge_index[1]
    pe = EP - N_EDGES
    src2d = jnp.concatenate([src, jnp.zeros((pe,), jnp.int32)]).reshape(NW, CPW, CH)
    dst2d = jnp.concatenate(
        [dst, jnp.full((pe,), N_NODES, jnp.int32)]).reshape(NW, CPW, CH)
    coef = jnp.concatenate([precomp.reshape(N_EDGES, KB), connection], axis=1)
    coef = jnp.concatenate([coef, jnp.zeros((pe, 8), f32)]).T
    zer24 = jnp.zeros((NPAD, 24), f32)
    zer8 = jnp.zeros((NPAD, 8), f32)

    Wm11 = _wprep(W11, 8, 3, 32, 1)
    Wm12 = _wprep(W12, 8, 3, 8, 3)
    Wm21 = _wprep(W21, 8, 3, 8, 3)
    Wm22 = _wprep(W22, 8, 3, 8, 3)
    Wm31 = _wprep(W31, 8, 1, 8, 3)
    Wm32 = _wprep(W32, 8, 1, 8, 1)
    S1pad = jnp.pad(S1.T, ((0, 0), (0, 16)))
    S2pad = jnp.zeros((24, 24), f32)
    for dd in range(3):
        S2pad = S2pad.at[dd * 8:(dd + 1) * 8, dd * 8:(dd + 1) * 8].set(S2.T)
    S3pad = jnp.concatenate([S3.T, jnp.zeros((16, 8), f32)], axis=0)
    lwb = jnp.concatenate([lin_w, lin_b[:, None]], axis=1)

    cfn = _normalize_cs()(coef)
    g32 = _make_gather(32)
    g24 = _make_gather(24)
    g8 = _make_gather(8)
    sc24 = _make_scatter(24)
    sc8 = _make_scatter(8)

    xs = g32(features, src2d)
    parts = sc24(_edge_mm(32, 1, 24)(xs, coef, Wm11), dst2d, zer24)
    h = _node_relu_sum(24)(parts, parts)
    xs = g24(h, src2d)
    parts = sc24(_edge_mm(8, 3, 24)(xs, cfn, Wm12), dst2d, zer24)
    x1 = _node_shortcut(24, 32)(parts, parts, features, S1pad)
    xs = g24(x1, src2d)
    parts = sc24(_edge_mm(8, 3, 24)(xs, cfn, Wm21), dst2d, zer24)
    h = _node_relu_sum(24)(parts, parts)
    xs = g24(h, src2d)
    parts = sc24(_edge_mm(8, 3, 24)(xs, cfn, Wm22), dst2d, zer24)
    x2 = _node_shortcut(24, 24)(parts, parts, x1, S2pad)
    xs = g24(x2, src2d)
    parts = sc8(_edge_mm(8, 3, 8)(xs, cfn, Wm31), dst2d, zer8)
    h = _node_relu_sum(8)(parts, parts)
    xs = g8(h, src2d)
    parts = sc8(_edge_mm(8, 1, 8)(xs, coef, Wm32), dst2d, zer8)
    y = _node_final()(parts, parts, x2, S3pad, lwb)
    return y[:, 0].reshape(NLAYERS, N_NODES // NLAYERS).T


# speedup vs baseline: 6.6514x; 1.2437x over previous
import functools

import jax
import jax.numpy as jnp
from jax import lax
from jax.experimental import pallas as pl
from jax.experimental.pallas import tpu as pltpu
from jax.experimental.pallas import tpu_sc as plsc

N_NODES = 50000
N_EDGES = 800000
NLAYERS = 5
KB = 6
DW = 32

NC, NS, CH = 2, 16, 128
NW = NC * NS
CPW = 196
EW = CPW * CH
EP = NW * EW
EP4 = EP // 4
NPAD = 51200
N4 = NPAD // 4
ROWS_PT = NPAD // NS

_mesh = plsc.VectorSubcoreMesh(core_axis_name="c", subcore_axis_name="s")
_sc_params = pltpu.CompilerParams(use_tc_tiling_on_sc=False)


def _make_gather(GK=7):
    GE = GK * CH
    NGRP = CPW // GK

    @functools.partial(
        pl.kernel,
        mesh=_mesh,
        compiler_params=_sc_params,
        out_type=jax.ShapeDtypeStruct((EP, DW), jnp.float32),
        scratch_types=[
            pltpu.VMEM((2, GK, CH), jnp.int32),
            pltpu.VMEM((2, GE, DW), jnp.float32),
            pltpu.SemaphoreType.DMA,
            pltpu.SemaphoreType.DMA,
        ],
    )
    def gk(table, src2d, out, idx2, buf2, sem_g, sem_o):
        cid = lax.axis_index("c")
        sid = lax.axis_index("s")
        wid = sid * NC + cid
        rows = src2d.at[wid]

        def load_idx(g, cur):
            pltpu.sync_copy(rows.at[pl.ds(g * GK, GK)], idx2.at[cur])

        def fire_group(cur):
            for j in range(GK):
                pltpu.async_copy(
                    table.at[idx2.at[cur].at[j]],
                    buf2.at[cur].at[pl.ds(j * CH, CH)], sem_g)

        load_idx(0, 0)
        fire_group(0)

        def body(t, carry):
            for cur in range(2):
                g = 2 * t + cur
                for j in range(GK):
                    pltpu.make_async_copy(
                        table.at[idx2.at[cur].at[j]],
                        buf2.at[cur].at[pl.ds(j * CH, CH)], sem_g).wait()

                @pl.when(g >= 1)
                def _():
                    pltpu.make_async_copy(
                        buf2.at[1 - cur], out.at[pl.ds(0, GE)], sem_o).wait()

                woff = pl.multiple_of(wid * EW + g * GE, GE)
                pltpu.async_copy(buf2.at[cur], out.at[pl.ds(woff, GE)], sem_o)

                @pl.when(g + 1 < NGRP)
                def _():
                    load_idx(g + 1, 1 - cur)
                    fire_group(1 - cur)
            return carry

        lax.fori_loop(0, NGRP // 2, body, 0)
        pltpu.make_async_copy(
            buf2.at[1], out.at[pl.ds(0, GE)], sem_o).wait()

    return gk


def _make_scatter(GK=2):
    GE = GK * CH
    NGRP = CPW // GK

    @functools.partial(
        pl.kernel,
        mesh=_mesh,
        compiler_params=_sc_params,
        out_type=jax.ShapeDtypeStruct((NC * NPAD, DW), jnp.float32),
        scratch_types=[
            pltpu.VMEM_SHARED((NPAD, DW), jnp.float32),
            pltpu.VMEM((2, GK, CH), jnp.int32),
            pltpu.VMEM((2, GE, DW), jnp.float32),
            pltpu.SemaphoreType.DMA,
        ],
    )
    def sk(msg, dst2d, zer, out, accum, idx2, mb2, sem_m):
        cid = lax.axis_index("c")
        sid = lax.axis_index("s")
        wid = sid * NC + cid
        r0 = pl.multiple_of(sid * ROWS_PT, ROWS_PT)
        pltpu.sync_copy(zer.at[pl.ds(r0, ROWS_PT)], accum.at[pl.ds(r0, ROWS_PT)])
        irows = dst2d.at[wid]
        pltpu.sync_copy(irows.at[pl.ds(0, GK)], idx2.at[0])
        plsc.subcore_barrier()

        def fire_load(g, cur):
            loff = pl.multiple_of(wid * EW + g * GE, GE)
            pltpu.async_copy(msg.at[pl.ds(loff, GE)], mb2.at[cur], sem_m)

        fire_load(0, 0)

        def body(t, carry):
            for cur in range(2):
                g = 2 * t + cur
                pltpu.make_async_copy(
                    msg.at[pl.ds(0, GE)], mb2.at[cur], sem_m).wait()

                @pl.when(g + 1 < NGRP)
                def _():
                    fire_load(g + 1, 1 - cur)
                    pltpu.sync_copy(
                        irows.at[pl.ds((g + 1) * GK, GK)], idx2.at[1 - cur])

                for j in range(GK):
                    pltpu.sync_copy(
                        mb2.at[cur].at[pl.ds(j * CH, CH)],
                        accum.at[idx2.at[cur].at[j]], add=True)
            return carry

        lax.fori_loop(0, NGRP // 2, body, 0)
        plsc.subcore_barrier()
        off = pl.multiple_of(cid * NPAD + r0, ROWS_PT)
        pltpu.sync_copy(accum.at[pl.ds(r0, ROWS_PT)], out.at[pl.ds(off, ROWS_PT)])

    return sk


def _normalize_cs(BE=4096):
    def body(cf_ref, out_ref):
        cf = cf_ref[...]
        c = cf[6:7, :]
        s = cf[7:8, :]
        nrm = jnp.sqrt(c * c + s * s) + 1e-8
        out_ref[...] = jnp.concatenate([cf[0:6, :], c / nrm, s / nrm], axis=0)

    return pl.pallas_call(
        body,
        grid=(EP // BE,),
        in_specs=[pl.BlockSpec((8, BE), lambda i: (0, i))],
        out_specs=pl.BlockSpec((8, BE), lambda i: (0, i)),
        out_shape=jax.ShapeDtypeStruct((8, EP), jnp.float32),
    )


def _edge_mm(Cin, din, O, BE=2048):
    KO = KB * O
    B4 = BE // 4

    def body(xs_ref, cf_ref, w_ref, out_ref):
        w = w_ref[...]
        outs = []
        for p in range(4):
            xt = xs_ref[:, p * DW:(p + 1) * DW].T
            cf = cf_ref[p * 8:(p + 1) * 8, :]
            if din == 3:
                c = cf[6:7, :]
                s = cf[7:8, :]
                x0 = xt[:Cin, :]
                x1 = xt[Cin:2 * Cin, :]
                x2 = xt[2 * Cin:3 * Cin, :]
                xt = jnp.concatenate(
                    [x0, x1 * c - x2 * s, x1 * s + x2 * c, xt[3 * Cin:, :]],
                    axis=0)
            tt = jnp.dot(w, xt, preferred_element_type=jnp.float32)
            m = cf[0:1, :] * tt[:O, :]
            for k in range(1, KB):
                m = m + cf[k:k + 1, :] * tt[k * O:(k + 1) * O, :]
            m32 = jnp.concatenate(
                [m, jnp.zeros((DW - O, B4), jnp.float32)], axis=0)
            outs.append(m32.T)
        out_ref[...] = jnp.concatenate(outs, axis=1)

    return pl.pallas_call(
        body,
        grid=(EP // BE,),
        in_specs=[
            pl.BlockSpec((B4, 128), lambda i: (i, 0)),
            pl.BlockSpec((32, B4), lambda i: (0, i)),
            pl.BlockSpec((KO, DW), lambda i: (0, 0)),
        ],
        out_specs=pl.BlockSpec((B4, 128), lambda i: (i, 0)),
        out_shape=jax.ShapeDtypeStruct((EP4, 128), jnp.float32),
    )


def _node_relu_sum(BN=32):
    def body(p0_ref, p1_ref, out_ref):
        out_ref[...] = jnp.maximum(p0_ref[...] + p1_ref[...], 0.0)

    return pl.pallas_call(
        body,
        grid=(N4 // BN,),
        in_specs=[
            pl.BlockSpec((BN, 128), lambda i: (i, 0)),
            pl.BlockSpec((BN, 128), lambda i: (i + N4 // BN, 0)),
        ],
        out_specs=pl.BlockSpec((BN, 128), lambda i: (i, 0)),
        out_shape=jax.ShapeDtypeStruct((N4, 128), jnp.float32),
    )


def _node_shortcut(BN=32):
    def body(p0_ref, p1_ref, xp_ref, s_ref, out_ref):
        s32 = s_ref[...]
        outs = []
        for p in range(4):
            xp = xp_ref[:, p * DW:(p + 1) * DW]
            sc = jnp.dot(xp, s32, preferred_element_type=jnp.float32)
            h = (p0_ref[:, p * DW:(p + 1) * DW]
                 + p1_ref[:, p * DW:(p + 1) * DW] + sc)
            outs.append(jnp.maximum(h, 0.0))
        out_ref[...] = jnp.concatenate(outs, axis=1)

    return pl.pallas_call(
        body,
        grid=(N4 // BN,),
        in_specs=[
            pl.BlockSpec((BN, 128), lambda i: (i, 0)),
            pl.BlockSpec((BN, 128), lambda i: (i + N4 // BN, 0)),
            pl.BlockSpec((BN, 128), lambda i: (i, 0)),
            pl.BlockSpec((DW, DW), lambda i: (0, 0)),
        ],
        out_specs=pl.BlockSpec((BN, 128), lambda i: (i, 0)),
        out_shape=jax.ShapeDtypeStruct((N4, 128), jnp.float32),
    )


def _node_final(BN=32):
    def body(p0_ref, p1_ref, xp_ref, s_ref, lwb_ref, out_ref):
        s32 = s_ref[...]
        outs = []
        for p in range(4):
            xp = xp_ref[:, p * DW:(p + 1) * DW]
            sc = jnp.dot(xp, s32, preferred_element_type=jnp.float32)
            x3 = jnp.maximum(
                p0_ref[:, p * DW:(p + 1) * DW]
                + p1_ref[:, p * DW:(p + 1) * DW] + sc, 0.0)
            y = (jnp.sum(x3[:, 0:8] * lwb_ref[0:1, 0:8], axis=1,
                         keepdims=True) + lwb_ref[0, 8])
            outs.append(jnp.maximum(y, 0.0))
        out_ref[...] = jnp.concatenate(outs, axis=1)

    return pl.pallas_call(
        body,
        grid=(N4 // BN,),
        in_specs=[
            pl.BlockSpec((BN, 128), lambda i: (i, 0)),
            pl.BlockSpec((BN, 128), lambda i: (i + N4 // BN, 0)),
            pl.BlockSpec((BN, 128), lambda i: (i, 0)),
            pl.BlockSpec((DW, DW), lambda i: (0, 0)),
            pl.BlockSpec((1, 9), lambda i: (0, 0)),
        ],
        out_specs=pl.BlockSpec((BN, 4), lambda i: (i, 0)),
        out_shape=jax.ShapeDtypeStruct((N4, 4), jnp.float32),
    )


def _wprep(W, Cout, dout, Cin, din):
    O = dout * Cout
    Wd = W.reshape(KB, Cout, dout, Cin, din).transpose(0, 2, 1, 4, 3)
    Wm = Wd.reshape(KB * O, din * Cin)
    return jnp.pad(Wm, ((0, 0), (0, DW - din * Cin)))


def kernel(features, edge_index, precomp, connection,
           W11, W12, S1, W21, W22, S2, W31, W32, S3, lin_w, lin_b):
    f32 = jnp.float32
    src = edge_index[0]
    dst = edge_index[1]
    pe = EP - N_EDGES
    src2d = jnp.concatenate([src, jnp.zeros((pe,), jnp.int32)]).reshape(NW, CPW, CH)
    dst2d = jnp.concatenate(
        [dst, jnp.full((pe,), N_NODES, jnp.int32)]).reshape(NW, CPW, CH)
    coef = jnp.concatenate([precomp.reshape(N_EDGES, KB), connection], axis=1)
    coef = jnp.concatenate([coef, jnp.zeros((pe, 8), f32)]).T
    zer = jnp.zeros((NPAD, DW), f32)

    Wm11 = _wprep(W11, 8, 3, 32, 1)
    Wm12 = _wprep(W12, 8, 3, 8, 3)
    Wm21 = _wprep(W21, 8, 3, 8, 3)
    Wm22 = _wprep(W22, 8, 3, 8, 3)
    Wm31 = _wprep(W31, 8, 1, 8, 3)
    Wm32 = _wprep(W32, 8, 1, 8, 1)
    S1p = jnp.zeros((DW, DW), f32).at[:32, 0:8].set(S1.T)
    S2p = jnp.zeros((DW, DW), f32)
    for dd in range(3):
        S2p = S2p.at[dd * 8:(dd + 1) * 8, dd * 8:(dd + 1) * 8].set(S2.T)
    S3p = jnp.zeros((DW, DW), f32).at[0:8, 0:8].set(S3.T)
    lwb = jnp.concatenate([lin_w, lin_b[:, None]], axis=1)

    cfn = _normalize_cs()(coef)
    cf32 = cfn.reshape(8, EP4, 4).transpose(2, 0, 1).reshape(32, EP4)

    feat32 = jnp.concatenate(
        [features, jnp.zeros((NPAD - N_NODES, 32), f32)])
    gat = _make_gather()
    sca = _make_scatter()

    def conv(table, Wm, Cin, din, O):
        xs = gat(table, src2d).reshape(EP4, 128)
        msg = _edge_mm(Cin, din, O)(xs, cf32, Wm).reshape(EP, DW)
        return sca(msg, dst2d, zer).reshape(NC * N4, 128)

    parts = conv(feat32, Wm11, 32, 1, 24)
    h = _node_relu_sum()(parts, parts)
    parts = conv(h.reshape(NPAD, DW), Wm12, 8, 3, 24)
    x1 = _node_shortcut()(parts, parts, feat32.reshape(N4, 128), S1p)
    parts = conv(x1.reshape(NPAD, DW), Wm21, 8, 3, 24)
    h = _node_relu_sum()(parts, parts)
    parts = conv(h.reshape(NPAD, DW), Wm22, 8, 3, 24)
    x2 = _node_shortcut()(parts, parts, x1, S2p)
    parts = conv(x2.reshape(NPAD, DW), Wm31, 8, 3, 8)
    h = _node_relu_sum()(parts, parts)
    parts = conv(h.reshape(NPAD, DW), Wm32, 8, 1, 8)
    y4 = _node_final()(parts, parts, x2, S3p, lwb)
    y = y4[:N_NODES // 4, :].reshape(N_NODES)
    return y.reshape(NLAYERS, N_NODES // NLAYERS).T


# speedup vs baseline: 8.7556x; 1.3163x over previous
import functools

import jax
import jax.numpy as jnp
from jax import lax
from jax.experimental import pallas as pl
from jax.experimental.pallas import tpu as pltpu
from jax.experimental.pallas import tpu_sc as plsc

N_NODES = 50000
N_EDGES = 800000
NLAYERS = 5
KB = 6
DW = 32

NC, NS, CH = 2, 16, 128
NW = NC * NS
CPW = 196
EW = CPW * CH
EP = NW * EW
EP4 = EP // 4
NPAD = 51200
N4 = NPAD // 4
ROWS_PT = NPAD // NS

_mesh = plsc.VectorSubcoreMesh(core_axis_name="c", subcore_axis_name="s")
_sc_params = pltpu.CompilerParams(use_tc_tiling_on_sc=False)


def _make_gather(GK=7):
    GE = GK * CH
    NGRP = CPW // GK

    @functools.partial(
        pl.kernel,
        mesh=_mesh,
        compiler_params=_sc_params,
        out_type=jax.ShapeDtypeStruct((EP, DW), jnp.float32),
        scratch_types=[
            pltpu.VMEM((2, GK, CH), jnp.int32),
            pltpu.VMEM((2, GE, DW), jnp.float32),
            pltpu.SemaphoreType.DMA,
            pltpu.SemaphoreType.DMA,
        ],
    )
    def gk(table, src2d, out, idx2, buf2, sem_g, sem_o):
        cid = lax.axis_index("c")
        sid = lax.axis_index("s")
        wid = sid * NC + cid
        rows = src2d.at[wid]

        def load_idx(g, cur):
            pltpu.sync_copy(rows.at[pl.ds(g * GK, GK)], idx2.at[cur])

        def fire_group(cur):
            for j in range(GK):
                pltpu.async_copy(
                    table.at[idx2.at[cur].at[j]],
                    buf2.at[cur].at[pl.ds(j * CH, CH)], sem_g)

        load_idx(0, 0)
        fire_group(0)

        def body(t, carry):
            for cur in range(2):
                g = 2 * t + cur
                for j in range(GK):
                    pltpu.make_async_copy(
                        table.at[idx2.at[cur].at[j]],
                        buf2.at[cur].at[pl.ds(j * CH, CH)], sem_g).wait()

                @pl.when(g >= 1)
                def _():
                    pltpu.make_async_copy(
                        buf2.at[1 - cur], out.at[pl.ds(0, GE)], sem_o).wait()

                woff = pl.multiple_of(wid * EW + g * GE, GE)
                pltpu.async_copy(buf2.at[cur], out.at[pl.ds(woff, GE)], sem_o)

                @pl.when(g + 1 < NGRP)
                def _():
                    load_idx(g + 1, 1 - cur)
                    fire_group(1 - cur)
            return carry

        lax.fori_loop(0, NGRP // 2, body, 0)
        pltpu.make_async_copy(
            buf2.at[1], out.at[pl.ds(0, GE)], sem_o).wait()

    return gk


def _make_scatter(GK=2):
    GE = GK * CH
    NGRP = CPW // GK

    @functools.partial(
        pl.kernel,
        mesh=_mesh,
        compiler_params=_sc_params,
        out_type=jax.ShapeDtypeStruct((NC * NPAD, DW), jnp.float32),
        scratch_types=[
            pltpu.VMEM_SHARED((NPAD, DW), jnp.float32),
            pltpu.VMEM((2, GK, CH), jnp.int32),
            pltpu.VMEM((2, GE, DW), jnp.float32),
            pltpu.SemaphoreType.DMA,
        ],
    )
    def sk(msg, dst2d, zer, out, accum, idx2, mb2, sem_m):
        cid = lax.axis_index("c")
        sid = lax.axis_index("s")
        wid = sid * NC + cid
        r0 = pl.multiple_of(sid * ROWS_PT, ROWS_PT)
        pltpu.sync_copy(zer.at[pl.ds(r0, ROWS_PT)], accum.at[pl.ds(r0, ROWS_PT)])
        irows = dst2d.at[wid]
        pltpu.sync_copy(irows.at[pl.ds(0, GK)], idx2.at[0])
        plsc.subcore_barrier()

        def fire_load(g, cur):
            loff = pl.multiple_of(wid * EW + g * GE, GE)
            pltpu.async_copy(msg.at[pl.ds(loff, GE)], mb2.at[cur], sem_m)

        fire_load(0, 0)

        def body(t, carry):
            for cur in range(2):
                g = 2 * t + cur
                pltpu.make_async_copy(
                    msg.at[pl.ds(0, GE)], mb2.at[cur], sem_m).wait()

                @pl.when(g + 1 < NGRP)
                def _():
                    fire_load(g + 1, 1 - cur)
                    pltpu.sync_copy(
                        irows.at[pl.ds((g + 1) * GK, GK)], idx2.at[1 - cur])

                for j in range(GK):
                    pltpu.sync_copy(
                        mb2.at[cur].at[pl.ds(j * CH, CH)],
                        accum.at[idx2.at[cur].at[j]], add=True)
            return carry

        lax.fori_loop(0, NGRP // 2, body, 0)
        plsc.subcore_barrier()
        off = pl.multiple_of(cid * NPAD + r0, ROWS_PT)
        pltpu.sync_copy(accum.at[pl.ds(r0, ROWS_PT)], out.at[pl.ds(off, ROWS_PT)])

    return sk


def _normalize_cs(BE=4096):
    def body(cf_ref, out_ref):
        cf = cf_ref[...]
        c = cf[6:7, :]
        s = cf[7:8, :]
        nrm = jnp.sqrt(c * c + s * s) + 1e-8
        out_ref[...] = jnp.concatenate([cf[0:6, :], c / nrm, s / nrm], axis=0)

    return pl.pallas_call(
        body,
        grid=(EP // BE,),
        in_specs=[pl.BlockSpec((8, BE), lambda i: (0, i))],
        out_specs=pl.BlockSpec((8, BE), lambda i: (0, i)),
        out_shape=jax.ShapeDtypeStruct((8, EP), jnp.float32),
    )


def _edge_mm(Cin, din, O, BE=2048):
    KO = KB * O
    B4 = BE // 4

    def body(xs_ref, cf_ref, w_ref, out_ref):
        w = w_ref[...]
        xt_all = xs_ref[...].T
        xt = jnp.concatenate(
            [xt_all[p * DW:(p + 1) * DW, :] for p in range(4)], axis=1)
        cf = jnp.concatenate(
            [cf_ref[p * 8:(p + 1) * 8, :] for p in range(4)], axis=1)
        if din == 3:
            c = cf[6:7, :]
            s = cf[7:8, :]
            x0 = xt[:Cin, :]
            x1 = xt[Cin:2 * Cin, :]
            x2 = xt[2 * Cin:3 * Cin, :]
            xt = jnp.concatenate(
                [x0, x1 * c - x2 * s, x1 * s + x2 * c, xt[3 * Cin:, :]],
                axis=0)
        tt = jnp.dot(w, xt, preferred_element_type=jnp.float32)
        m = cf[0:1, :] * tt[:O, :]
        for k in range(1, KB):
            m = m + cf[k:k + 1, :] * tt[k * O:(k + 1) * O, :]
        m32 = jnp.concatenate(
            [m, jnp.zeros((DW - O, BE), jnp.float32)], axis=0)
        m128 = jnp.concatenate(
            [m32[:, p * B4:(p + 1) * B4] for p in range(4)], axis=0)
        out_ref[...] = m128.T

    return pl.pallas_call(
        body,
        grid=(EP // BE,),
        in_specs=[
            pl.BlockSpec((B4, 128), lambda i: (i, 0)),
            pl.BlockSpec((32, B4), lambda i: (0, i)),
            pl.BlockSpec((KO, DW), lambda i: (0, 0)),
        ],
        out_specs=pl.BlockSpec((B4, 128), lambda i: (i, 0)),
        out_shape=jax.ShapeDtypeStruct((EP4, 128), jnp.float32),
    )


def _node_relu_sum(BN=32):
    def body(p0_ref, p1_ref, out_ref):
        out_ref[...] = jnp.maximum(p0_ref[...] + p1_ref[...], 0.0)

    return pl.pallas_call(
        body,
        grid=(N4 // BN,),
        in_specs=[
            pl.BlockSpec((BN, 128), lambda i: (i, 0)),
            pl.BlockSpec((BN, 128), lambda i: (i + N4 // BN, 0)),
        ],
        out_specs=pl.BlockSpec((BN, 128), lambda i: (i, 0)),
        out_shape=jax.ShapeDtypeStruct((N4, 128), jnp.float32),
    )


def _node_shortcut(BN=32):
    def body(p0_ref, p1_ref, xp_ref, s_ref, out_ref):
        s32 = s_ref[...]
        outs = []
        for p in range(4):
            xp = xp_ref[:, p * DW:(p + 1) * DW]
            sc = jnp.dot(xp, s32, preferred_element_type=jnp.float32)
            h = (p0_ref[:, p * DW:(p + 1) * DW]
                 + p1_ref[:, p * DW:(p + 1) * DW] + sc)
            outs.append(jnp.maximum(h, 0.0))
        out_ref[...] = jnp.concatenate(outs, axis=1)

    return pl.pallas_call(
        body,
        grid=(N4 // BN,),
        in_specs=[
            pl.BlockSpec((BN, 128), lambda i: (i, 0)),
            pl.BlockSpec((BN, 128), lambda i: (i + N4 // BN, 0)),
            pl.BlockSpec((BN, 128), lambda i: (i, 0)),
            pl.BlockSpec((DW, DW), lambda i: (0, 0)),
        ],
        out_specs=pl.BlockSpec((BN, 128), lambda i: (i, 0)),
        out_shape=jax.ShapeDtypeStruct((N4, 128), jnp.float32),
    )


def _node_final(BN=32):
    def body(p0_ref, p1_ref, xp_ref, s_ref, lwb_ref, out_ref):
        s32 = s_ref[...]
        outs = []
        for p in range(4):
            xp = xp_ref[:, p * DW:(p + 1) * DW]
            sc = jnp.dot(xp, s32, preferred_element_type=jnp.float32)
            x3 = jnp.maximum(
                p0_ref[:, p * DW:(p + 1) * DW]
                + p1_ref[:, p * DW:(p + 1) * DW] + sc, 0.0)
            y = (jnp.sum(x3[:, 0:8] * lwb_ref[0:1, 0:8], axis=1,
                         keepdims=True) + lwb_ref[0, 8])
            outs.append(jnp.maximum(y, 0.0))
        out_ref[...] = jnp.concatenate(outs, axis=1)

    return pl.pallas_call(
        body,
        grid=(N4 // BN,),
        in_specs=[
            pl.BlockSpec((BN, 128), lambda i: (i, 0)),
            pl.BlockSpec((BN, 128), lambda i: (i + N4 // BN, 0)),
            pl.BlockSpec((BN, 128), lambda i: (i, 0)),
            pl.BlockSpec((DW, DW), lambda i: (0, 0)),
            pl.BlockSpec((1, 9), lambda i: (0, 0)),
        ],
        out_specs=pl.BlockSpec((BN, 4), lambda i: (i, 0)),
        out_shape=jax.ShapeDtypeStruct((N4, 4), jnp.float32),
    )


def _wprep(W, Cout, dout, Cin, din):
    O = dout * Cout
    Wd = W.reshape(KB, Cout, dout, Cin, din).transpose(0, 2, 1, 4, 3)
    Wm = Wd.reshape(KB * O, din * Cin)
    return jnp.pad(Wm, ((0, 0), (0, DW - din * Cin)))


def kernel(features, edge_index, precomp, connection,
           W11, W12, S1, W21, W22, S2, W31, W32, S3, lin_w, lin_b):
    f32 = jnp.float32
    src = edge_index[0]
    dst = edge_index[1]
    pe = EP - N_EDGES
    src2d = jnp.concatenate([src, jnp.zeros((pe,), jnp.int32)]).reshape(NW, CPW, CH)
    dst2d = jnp.concatenate(
        [dst, jnp.full((pe,), N_NODES, jnp.int32)]).reshape(NW, CPW, CH)
    coef = jnp.concatenate([precomp.reshape(N_EDGES, KB), connection], axis=1)
    coef = jnp.concatenate([coef, jnp.zeros((pe, 8), f32)]).T
    zer = jnp.zeros((NPAD, DW), f32)

    Wm11 = _wprep(W11, 8, 3, 32, 1)
    Wm12 = _wprep(W12, 8, 3, 8, 3)
    Wm21 = _wprep(W21, 8, 3, 8, 3)
    Wm22 = _wprep(W22, 8, 3, 8, 3)
    Wm31 = _wprep(W31, 8, 1, 8, 3)
    Wm32 = _wprep(W32, 8, 1, 8, 1)
    S1p = jnp.zeros((DW, DW), f32).at[:32, 0:8].set(S1.T)
    S2p = jnp.zeros((DW, DW), f32)
    for dd in range(3):
        S2p = S2p.at[dd * 8:(dd + 1) * 8, dd * 8:(dd + 1) * 8].set(S2.T)
    S3p = jnp.zeros((DW, DW), f32).at[0:8, 0:8].set(S3.T)
    lwb = jnp.concatenate([lin_w, lin_b[:, None]], axis=1)

    cfn = _normalize_cs()(coef)
    cf32 = cfn.reshape(8, EP4, 4).transpose(2, 0, 1).reshape(32, EP4)

    feat32 = jnp.concatenate(
        [features, jnp.zeros((NPAD - N_NODES, 32), f32)])
    gat = _make_gather()
    sca = _make_scatter()

    def conv(table, Wm, Cin, din, O):
        xs = gat(table, src2d).reshape(EP4, 128)
        msg = _edge_mm(Cin, din, O)(xs, cf32, Wm).reshape(EP, DW)
        return sca(msg, dst2d, zer).reshape(NC * N4, 128)

    parts = conv(feat32, Wm11, 32, 1, 24)
    h = _node_relu_sum()(parts, parts)
    parts = conv(h.reshape(NPAD, DW), Wm12, 8, 3, 24)
    x1 = _node_shortcut()(parts, parts, feat32.reshape(N4, 128), S1p)
    parts = conv(x1.reshape(NPAD, DW), Wm21, 8, 3, 24)
    h = _node_relu_sum()(parts, parts)
    parts = conv(h.reshape(NPAD, DW), Wm22, 8, 3, 24)
    x2 = _node_shortcut()(parts, parts, x1, S2p)
    parts = conv(x2.reshape(NPAD, DW), Wm31, 8, 3, 8)
    h = _node_relu_sum()(parts, parts)
    parts = conv(h.reshape(NPAD, DW), Wm32, 8, 1, 8)
    y4 = _node_final()(parts, parts, x2, S3p, lwb)
    y = y4[:N_NODES // 4, :].reshape(N_NODES)
    return y.reshape(NLAYERS, N_NODES // NLAYERS).T
